# baseline (speedup 1.0000x reference)
import functools

import jax
import jax.numpy as jnp
from jax import lax
from jax.experimental import pallas as pl
from jax.experimental.pallas import tpu as pltpu
from jax.experimental.pallas import tpu_sc as plsc

B, T, D, M, TOP_K = 1, 2048, 768, 16384, 8

TBLK = 512
MBLK = 2048
NT = T // TBLK
NM = M // MBLK

NEG_INF = float("-inf")


SPLIT = 1
SUBW = MBLK // SPLIT


def _topk_scan(c, base):
  w = c.shape[1]
  rows = c.shape[0]
  iota = lax.broadcasted_iota(jnp.int32, c.shape, 1)
  i8 = lax.broadcasted_iota(jnp.int32, (rows, TOP_K), 1)
  vals_arr = jnp.full((rows, TOP_K), NEG_INF, jnp.float32)
  idx_arr = jnp.zeros((rows, TOP_K), jnp.int32)
  for it in range(TOP_K):
    m = jnp.max(c, axis=1, keepdims=True)
    pos = jnp.min(jnp.where(c == m, iota, w), axis=1, keepdims=True)
    c = jnp.where(iota == pos, NEG_INF, c)
    vals_arr = jnp.where(i8 == it, m, vals_arr)
    idx_arr = jnp.where(i8 == it, pos + base, idx_arr)
  return vals_arr, idx_arr


def _before(va, ia, vb, ib):
  return (va > vb) | ((va == vb) & (ia < ib))


def _merge8(va, ia, vb, ib):
  rvb = jnp.concatenate([vb[:, 7 - i:8 - i] for i in range(TOP_K)], 1)
  rib = jnp.concatenate([ib[:, 7 - i:8 - i] for i in range(TOP_K)], 1)
  keep_a = _before(va, ia, rvb, rib)
  wv = jnp.where(keep_a, va, rvb)
  wi = jnp.where(keep_a, ia, rib)
  for d in (4, 2, 1):
    pv, pi = [], []
    for g in range(0, TOP_K, 2 * d):
      xv, yv = wv[:, g:g + d], wv[:, g + d:g + 2 * d]
      xi, yi = wi[:, g:g + d], wi[:, g + d:g + 2 * d]
      sel = _before(xv, xi, yv, yi)
      pv += [jnp.where(sel, xv, yv), jnp.where(sel, yv, xv)]
      pi += [jnp.where(sel, xi, yi), jnp.where(sel, yi, xi)]
    wv = jnp.concatenate(pv, 1)
    wi = jnp.concatenate(pi, 1)
  return wv, wi


TSPLIT = 1
TROWS = TBLK // TSPLIT


def _topk_block(c, j):
  parts = [
      _topk_scan(c[s * TROWS:(s + 1) * TROWS, :], j * MBLK)
      for s in range(TSPLIT)
  ]
  vs = jnp.concatenate([p[0] for p in parts], 0)
  is_ = jnp.concatenate([p[1] for p in parts], 0)
  return vs, is_


def _tc_body(nt, x_ref, w_ref, vals_ref, idx_ref, loss_ref, c_scr, run_v,
             run_i):
  j = pl.program_id(0)
  i = pl.program_id(1)
  tp = jax.lax.rem(i + nt - 1, nt)
  jp = jnp.where(i == 0, j - 1, j)

  @pl.when((j < NM) | (i == 0))
  def _():
    c = c_scr[...]
    newv, newi = _topk_block(c, jp)
    cm = jax.lax.dot_general(
        x_ref[...], w_ref[...],
        dimension_numbers=(((1,), (1,)), ((), ())),
        preferred_element_type=jnp.float32)
    c_scr[...] = cm
    mv, mi = _merge8(run_v[tp], run_i[tp], newv, newi)
    first = jp == 0
    run_v[tp] = jnp.where(first, newv, mv)
    run_i[tp] = jnp.where(first, newi, mi)

  @pl.when(j == NM)
  def _():
    rv = run_v[tp]
    ri = run_i[tp]
    col = lax.broadcasted_iota(jnp.int32, (TBLK, TOP_K * 16), 1) // 16
    vb = jnp.zeros((TBLK, TOP_K * 16), jnp.float32)
    for k in range(TOP_K):
      vb = jnp.where(col == k, rv[:, k][:, None], vb)
    vals_ref[...] = vb
    idx_ref[...] = ri
    part = jnp.sum(jnp.abs(rv), keepdims=True).reshape(1, 1)

    @pl.when(i == 0)
    def _():
      loss_ref[...] = part

    @pl.when(i > 0)
    def _():
      loss_ref[...] = loss_ref[...] + part


def _encode_topk(x2d, w_enc):
  t = x2d.shape[0]
  nt = t // TBLK
  return pl.pallas_call(
      functools.partial(_tc_body, nt),
      grid=(NM + 1, nt),
      in_specs=[
          pl.BlockSpec((TBLK, D), lambda j, i: (i, 0)),
          pl.BlockSpec((MBLK, D), lambda j, i: (jnp.minimum(j, NM - 1), 0)),
      ],
      out_specs=[
          pl.BlockSpec((TBLK, TOP_K * 16),
                       lambda j, i: ((i + nt - 1) % nt, 0)),
          pl.BlockSpec((TBLK, TOP_K),
                       lambda j, i: ((i + nt - 1) % nt, 0)),
          pl.BlockSpec((1, 1), lambda j, i: (0, 0)),
      ],
      out_shape=[
          jax.ShapeDtypeStruct((t, TOP_K * 16), jnp.float32),
          jax.ShapeDtypeStruct((t, TOP_K), jnp.int32),
          jax.ShapeDtypeStruct((1, 1), jnp.float32),
      ],
      scratch_shapes=[
          pltpu.VMEM((TBLK, MBLK), jnp.float32),
          pltpu.VMEM((nt, TBLK, TOP_K), jnp.float32),
          pltpu.VMEM((nt, TBLK, TOP_K), jnp.int32),
      ],
      compiler_params=pltpu.CompilerParams(
          dimension_semantics=("arbitrary", "arbitrary")),
  )(x2d, w_enc)


NC, NS = 2, 16
NW = NC * NS
TOK_W = T // NW
CHUNK_T = 8
ROWS_C = CHUNK_T * TOP_K
NCHUNK = TOK_W // CHUNK_T
LANES = 16
DV = D // LANES


DUNROLL = 8
DGRP = DV // DUNROLL


def _sc_body(tok_w, nchunk, w_hbm, idx_hbm, vals_hbm, out_hbm, idx_v, vals_v,
             rows2_v, out_v, sem0, sem1):
  wid = lax.axis_index("s") * NC + lax.axis_index("c")
  pltpu.sync_copy(idx_hbm.at[pl.ds(wid * nchunk, nchunk)], idx_v)
  pltpu.sync_copy(vals_hbm.at[pl.ds(wid * tok_w, tok_w)], vals_v)
  sems = (sem0, sem1)

  def start(c, b):
    pltpu.async_copy(w_hbm.at[idx_v.at[c]], rows2_v.at[b], sems[b])

  def do_chunk(c, b):
    pltpu.make_async_copy(w_hbm.at[idx_v.at[c]], rows2_v.at[b],
                          sems[b]).wait()

    def tok_body(t, _):
      vks = [vals_v[c * CHUNK_T + t, pl.ds(k * LANES, LANES)]
             for k in range(TOP_K)]

      def d_body(g, _):
        for du in range(DUNROLL):
          sl = pl.ds(g * (DUNROLL * LANES) + du * LANES, LANES)
          acc = vks[0] * rows2_v[b, t * TOP_K, sl]
          for k in range(1, TOP_K):
            acc += vks[k] * rows2_v[b, t * TOP_K + k, sl]
          out_v[t, sl] = acc
        return 0

      lax.fori_loop(0, DGRP, d_body, 0)
      return 0

    lax.fori_loop(0, CHUNK_T, tok_body, 0)
    pltpu.sync_copy(out_v,
                    out_hbm.at[pl.ds(wid * tok_w + c * CHUNK_T, CHUNK_T)])

  start(0, 0)
  for h in range(nchunk // 2):
    start(2 * h + 1, 1)
    do_chunk(2 * h, 0)
    if h < nchunk // 2 - 1:
      start(2 * h + 2, 0)
    do_chunk(2 * h + 1, 1)


def _decode(w_unified, idx2d, vals_bcast):
  t = vals_bcast.shape[0]
  tok_w = t // NW
  nchunk = tok_w // CHUNK_T
  mesh = plsc.VectorSubcoreMesh(core_axis_name="c", subcore_axis_name="s")
  f = pl.kernel(
      functools.partial(_sc_body, tok_w, nchunk),
      out_type=jax.ShapeDtypeStruct((t, D), jnp.float32),
      mesh=mesh,
      scratch_types=[
          pltpu.VMEM((nchunk, ROWS_C), jnp.int32),
          pltpu.VMEM((tok_w, TOP_K * 16), jnp.float32),
          pltpu.VMEM((2, ROWS_C, D), jnp.float32),
          pltpu.VMEM((CHUNK_T, D), jnp.float32),
          pltpu.SemaphoreType.DMA,
          pltpu.SemaphoreType.DMA,
      ],
  )
  return f(w_unified, idx2d, vals_bcast)


HALF = T // 2


@jax.jit
def kernel(x, W_enc, W_unified):
  x2d = x.reshape(T, D)
  offs, losses = [], []
  for hf in range(2):
    xh = x2d[hf * HALF:(hf + 1) * HALF]
    vb, idx, loss = _encode_topk(xh, W_enc)
    idx2d = idx.reshape(HALF * TOP_K // ROWS_C, ROWS_C)
    offs.append(_decode(W_unified, idx2d, vb))
    losses.append(loss[0, 0])
  offset = jnp.concatenate(offs, 0)
  sparsity_loss = (losses[0] + losses[1]) / (B * T * M)
  return offset.reshape(B, T, D), sparsity_loss


# speedup vs baseline: 1.0155x; 1.0155x over previous
import functools

import jax
import jax.numpy as jnp
from jax import lax
from jax.experimental import pallas as pl
from jax.experimental.pallas import tpu as pltpu
from jax.experimental.pallas import tpu_sc as plsc

B, T, D, M, TOP_K = 1, 2048, 768, 16384, 8

TBLK = 512
MBLK = 2048
NT = T // TBLK
NM = M // MBLK

NEG_INF = float("-inf")


SPLIT = 1
SUBW = MBLK // SPLIT


def _topk_scan(c, base):
  w = c.shape[1]
  rows = c.shape[0]
  iota = lax.broadcasted_iota(jnp.int32, c.shape, 1)
  i8 = lax.broadcasted_iota(jnp.int32, (rows, TOP_K), 1)
  vals_arr = jnp.full((rows, TOP_K), NEG_INF, jnp.float32)
  idx_arr = jnp.zeros((rows, TOP_K), jnp.int32)
  for it in range(TOP_K):
    m = jnp.max(c, axis=1, keepdims=True)
    pos = jnp.min(jnp.where(c == m, iota, w), axis=1, keepdims=True)
    c = jnp.where(iota == pos, NEG_INF, c)
    vals_arr = jnp.where(i8 == it, m, vals_arr)
    idx_arr = jnp.where(i8 == it, pos + base, idx_arr)
  return vals_arr, idx_arr


def _before(va, ia, vb, ib):
  return (va > vb) | ((va == vb) & (ia < ib))


def _merge8(va, ia, vb, ib):
  rvb = jnp.concatenate([vb[:, 7 - i:8 - i] for i in range(TOP_K)], 1)
  rib = jnp.concatenate([ib[:, 7 - i:8 - i] for i in range(TOP_K)], 1)
  keep_a = _before(va, ia, rvb, rib)
  wv = jnp.where(keep_a, va, rvb)
  wi = jnp.where(keep_a, ia, rib)
  for d in (4, 2, 1):
    pv, pi = [], []
    for g in range(0, TOP_K, 2 * d):
      xv, yv = wv[:, g:g + d], wv[:, g + d:g + 2 * d]
      xi, yi = wi[:, g:g + d], wi[:, g + d:g + 2 * d]
      sel = _before(xv, xi, yv, yi)
      pv += [jnp.where(sel, xv, yv), jnp.where(sel, yv, xv)]
      pi += [jnp.where(sel, xi, yi), jnp.where(sel, yi, xi)]
    wv = jnp.concatenate(pv, 1)
    wi = jnp.concatenate(pi, 1)
  return wv, wi


TSPLIT = 1
TROWS = TBLK // TSPLIT


def _topk_block(c, j):
  parts = [
      _topk_scan(c[s * TROWS:(s + 1) * TROWS, :], j * MBLK)
      for s in range(TSPLIT)
  ]
  vs = jnp.concatenate([p[0] for p in parts], 0)
  is_ = jnp.concatenate([p[1] for p in parts], 0)
  return vs, is_


def _tc_body(x_ref, w_ref, vals_ref, idx_ref, loss_ref, c_scr, run_v, run_i):
  j = pl.program_id(0)
  i = pl.program_id(1)
  tp = jax.lax.rem(i + NT - 1, NT)
  jp = jnp.where(i == 0, j - 1, j)

  @pl.when((j < NM) | (i == 0))
  def _():
    c = c_scr[...]
    newv, newi = _topk_block(c, jp)
    cm = jax.lax.dot_general(
        x_ref[...], w_ref[...],
        dimension_numbers=(((1,), (1,)), ((), ())),
        preferred_element_type=jnp.float32)
    c_scr[...] = cm
    mv, mi = _merge8(run_v[tp], run_i[tp], newv, newi)
    first = jp == 0
    run_v[tp] = jnp.where(first, newv, mv)
    run_i[tp] = jnp.where(first, newi, mi)

  @pl.when(j == NM)
  def _():
    rv = run_v[tp]
    ri = run_i[tp]
    col = lax.broadcasted_iota(jnp.int32, (TBLK, TOP_K * 16), 1) // 16
    vb = jnp.zeros((TBLK, TOP_K * 16), jnp.float32)
    for k in range(TOP_K):
      vb = jnp.where(col == k, rv[:, k][:, None], vb)
    vals_ref[...] = vb
    idx_ref[...] = ri
    part = jnp.sum(jnp.abs(rv), keepdims=True).reshape(1, 1)

    @pl.when(i == 0)
    def _():
      loss_ref[...] = part

    @pl.when(i > 0)
    def _():
      loss_ref[...] = loss_ref[...] + part


def _encode_topk(x2d, w_enc):
  return pl.pallas_call(
      _tc_body,
      grid=(NM + 1, NT),
      in_specs=[
          pl.BlockSpec((TBLK, D), lambda j, i: (i, 0)),
          pl.BlockSpec((MBLK, D), lambda j, i: (jnp.minimum(j, NM - 1), 0)),
      ],
      out_specs=[
          pl.BlockSpec((TBLK, TOP_K * 16),
                       lambda j, i: ((i + NT - 1) % NT, 0)),
          pl.BlockSpec((TBLK, TOP_K),
                       lambda j, i: ((i + NT - 1) % NT, 0)),
          pl.BlockSpec((1, 1), lambda j, i: (0, 0)),
      ],
      out_shape=[
          jax.ShapeDtypeStruct((T, TOP_K * 16), jnp.float32),
          jax.ShapeDtypeStruct((T, TOP_K), jnp.int32),
          jax.ShapeDtypeStruct((1, 1), jnp.float32),
      ],
      scratch_shapes=[
          pltpu.VMEM((TBLK, MBLK), jnp.float32),
          pltpu.VMEM((NT, TBLK, TOP_K), jnp.float32),
          pltpu.VMEM((NT, TBLK, TOP_K), jnp.int32),
      ],
      compiler_params=pltpu.CompilerParams(
          dimension_semantics=("arbitrary", "arbitrary")),
  )(x2d, w_enc)


NC, NS = 2, 16
NW = NC * NS
TOK_W = T // NW
CHUNK_T = 8
ROWS_C = CHUNK_T * TOP_K
NCHUNK = TOK_W // CHUNK_T
LANES = 16
DV = D // LANES


DUNROLL = 8
DGRP = DV // DUNROLL


def _sc_body(w_hbm, idx_hbm, vals_hbm, out_hbm, idx_v, vals_v, rows2_v,
             out_v, sem0, sem1):
  wid = lax.axis_index("s") * NC + lax.axis_index("c")
  pltpu.sync_copy(idx_hbm.at[pl.ds(wid * NCHUNK, NCHUNK)], idx_v)
  pltpu.sync_copy(vals_hbm.at[pl.ds(wid * TOK_W, TOK_W)], vals_v)
  sems = (sem0, sem1)

  def start(c, b):
    pltpu.async_copy(w_hbm.at[idx_v.at[c]], rows2_v.at[b], sems[b])

  def do_chunk(c, b):
    pltpu.make_async_copy(w_hbm.at[idx_v.at[c]], rows2_v.at[b],
                          sems[b]).wait()

    def tok_body(t, _):
      vks = [vals_v[c * CHUNK_T + t, pl.ds(k * LANES, LANES)]
             for k in range(TOP_K)]

      def d_body(g, _):
        for du in range(DUNROLL):
          sl = pl.ds(g * (DUNROLL * LANES) + du * LANES, LANES)
          acc = vks[0] * rows2_v[b, t * TOP_K, sl]
          for k in range(1, TOP_K):
            acc += vks[k] * rows2_v[b, t * TOP_K + k, sl]
          out_v[t, sl] = acc
        return 0

      lax.fori_loop(0, DGRP, d_body, 0)
      return 0

    lax.fori_loop(0, CHUNK_T, tok_body, 0)
    pltpu.sync_copy(out_v,
                    out_hbm.at[pl.ds(wid * TOK_W + c * CHUNK_T, CHUNK_T)])

  start(0, 0)
  for h in range(NCHUNK // 2):
    start(2 * h + 1, 1)
    do_chunk(2 * h, 0)
    if h < NCHUNK // 2 - 1:
      start(2 * h + 2, 0)
    do_chunk(2 * h + 1, 1)


def _decode(w_unified, idx2d, vals_flat):
  mesh = plsc.VectorSubcoreMesh(core_axis_name="c", subcore_axis_name="s")
  f = pl.kernel(
      _sc_body,
      out_type=jax.ShapeDtypeStruct((T, D), jnp.float32),
      mesh=mesh,
      scratch_types=[
          pltpu.VMEM((NCHUNK, ROWS_C), jnp.int32),
          pltpu.VMEM((TOK_W, TOP_K * 16), jnp.float32),
          pltpu.VMEM((2, ROWS_C, D), jnp.float32),
          pltpu.VMEM((CHUNK_T, D), jnp.float32),
          pltpu.SemaphoreType.DMA,
          pltpu.SemaphoreType.DMA,
      ],
  )
  return f(w_unified, idx2d, vals_flat)


@jax.jit
def kernel(x, W_enc, W_unified):
  x2d = x.reshape(T, D)
  vb, idx, loss = _encode_topk(x2d, W_enc)
  idx2d = idx.reshape(T * TOP_K // ROWS_C, ROWS_C)
  offset = _decode(W_unified, idx2d, vb)
  sparsity_loss = loss[0, 0] / (B * T * M)
  return offset.reshape(B, T, D), sparsity_loss
